# baseline (speedup 1.0000x reference)
import functools

import jax
import jax.numpy as jnp
from jax import lax
from jax.experimental import pallas as pl
from jax.experimental.pallas import tpu as pltpu
from jax.experimental.pallas import tpu_sc as plsc

N_NODES = 100000
IN_CH = 8
OUT_CH = 32

NC = 2
NS = 16
NW = NC * NS
SUB = 128
K = 16
CPW = 50
E_PAD = NW * CPW * K * SUB
ROWS_PER_TILE = 6272
N_PAD = NS * ROWS_PER_TILE
XS = 6256
XS_LAST = N_NODES - (NS - 1) * XS


def _sc_mesh():
    return plsc.VectorSubcoreMesh(core_axis_name="c", subcore_axis_name="s")


@functools.partial(
    pl.kernel,
    out_type=jax.ShapeDtypeStruct((NC, N_PAD, IN_CH), jnp.float32),
    mesh=_sc_mesh(),
    scratch_types=[
        pltpu.VMEM_SHARED((N_PAD, IN_CH), jnp.float32),
        pltpu.VMEM_SHARED((N_PAD, IN_CH), jnp.float32),
        pltpu.VMEM((2, K, 3, SUB), jnp.int32),
        pltpu.VMEM((K, SUB, IN_CH), jnp.float32),
        pltpu.SemaphoreType.DMA,
        pltpu.SemaphoreType.DMA,
        pltpu.SemaphoreType.DMA,
    ],
    compiler_params=pltpu.CompilerParams(needs_layout_passes=False,
                                         use_tc_tiling_on_sc=False),
)
def _sc_aggregate(x_hbm, ed_hbm, zero_hbm, out_hbm,
                  x_sh, agg_sh, edata_v, rows_v, sem_e, sem_g, sem_s):
    c = lax.axis_index("c")
    s = lax.axis_index("s")
    wid = c * NS + s

    pltpu.sync_copy(zero_hbm, agg_sh.at[pl.ds(s * ROWS_PER_TILE, ROWS_PER_TILE)])

    @pl.when(s < NS - 1)
    def _():
        pltpu.sync_copy(x_hbm.at[pl.ds(s * XS, XS)], x_sh.at[pl.ds(s * XS, XS)])

    @pl.when(s == NS - 1)
    def _():
        pltpu.sync_copy(x_hbm.at[pl.ds(s * XS, XS_LAST)],
                        x_sh.at[pl.ds(s * XS, XS_LAST)])

    plsc.subcore_barrier()

    iota = lax.broadcasted_iota(jnp.int32, (16,), 0)
    cidx = jnp.bitwise_and(iota, 7)
    half = jnp.right_shift(iota, 3)
    two = jnp.full((16,), 2, jnp.int32)
    row_base = wid * (CPW * K)

    pltpu.async_copy(ed_hbm.at[pl.ds(row_base, K)], edata_v.at[0], sem_e)

    def chunk_body(g, carry):
        p = jnp.bitwise_and(g, 1)
        q = 1 - p
        row0 = row_base + g * K

        pltpu.make_async_copy(ed_hbm.at[pl.ds(row0, K)], edata_v.at[p],
                              sem_e).wait()

        @pl.when(g < CPW - 1)
        def _():
            pltpu.async_copy(ed_hbm.at[pl.ds(row0 + K, K)], edata_v.at[q],
                             sem_e)

        gathers = [pltpu.async_copy(x_sh.at[edata_v.at[p, j, 0]],
                                    rows_v.at[j], sem_g)
                   for j in range(K)]
        for h in gathers:
            h.wait()

        pv = jnp.full((16,), p, jnp.int32)

        def mul_body(i):
            jv = jnp.full((16,), jnp.right_shift(i, 6), jnp.int32)
            ridx = half + 2 * jnp.bitwise_and(i, 63)
            w = plsc.bitcast(
                plsc.load_gather(edata_v, [pv, jv, two, ridx]), jnp.float32)
            rows16 = plsc.load_gather(rows_v, [jv, ridx, cidx])
            plsc.store_scatter(rows_v, [jv, ridx, cidx], rows16 * w)

        plsc.parallel_loop(0, K * (SUB // 2), 1, unroll=8)(mul_body)

        scatters = [pltpu.async_copy(rows_v.at[j],
                                     agg_sh.at[edata_v.at[p, j, 1]],
                                     sem_s, add=True) for j in range(K)]
        for h in scatters:
            h.wait()
        return carry

    lax.fori_loop(0, CPW, chunk_body, 0)

    plsc.subcore_barrier()
    pltpu.sync_copy(agg_sh.at[pl.ds(s * ROWS_PER_TILE, ROWS_PER_TILE)],
                    out_hbm.at[c, pl.ds(s * ROWS_PER_TILE, ROWS_PER_TILE)])


def _tc_body(p_ref, x_ref, wr_ref, wo_ref, br_ref, bo_ref, o_ref):
    agg = p_ref[0] + p_ref[1]
    o_ref[...] = (
        jnp.dot(agg, wr_ref[...], preferred_element_type=jnp.float32)
        + jnp.dot(x_ref[...], wo_ref[...], preferred_element_type=jnp.float32)
        + br_ref[...] + bo_ref[...]
    )


def _tc_update(partials, x, W_rel, W_root, b_rel, b_root):
    R = 2000
    n_blocks = N_NODES // R
    return pl.pallas_call(
        _tc_body,
        grid=(n_blocks,),
        in_specs=[
            pl.BlockSpec((NC, R, IN_CH), lambda i: (0, i, 0)),
            pl.BlockSpec((R, IN_CH), lambda i: (i, 0)),
            pl.BlockSpec((IN_CH, OUT_CH), lambda i: (0, 0)),
            pl.BlockSpec((IN_CH, OUT_CH), lambda i: (0, 0)),
            pl.BlockSpec((1, OUT_CH), lambda i: (0, 0)),
            pl.BlockSpec((1, OUT_CH), lambda i: (0, 0)),
        ],
        out_specs=pl.BlockSpec((R, OUT_CH), lambda i: (i, 0)),
        out_shape=jax.ShapeDtypeStruct((N_NODES, OUT_CH), jnp.float32),
    )(partials, x, W_rel, W_root, b_rel.reshape(1, OUT_CH),
      b_root.reshape(1, OUT_CH))


def kernel(x, edge_index, edge_weight, W_rel, b_rel, W_root, b_root):
    e = edge_index.shape[1]
    pad = E_PAD - e
    src = jnp.concatenate(
        [edge_index[0].astype(jnp.int32), jnp.zeros((pad,), jnp.int32)])
    dst = jnp.concatenate(
        [edge_index[1].astype(jnp.int32), jnp.zeros((pad,), jnp.int32)])
    ewb = jnp.concatenate(
        [lax.bitcast_convert_type(edge_weight.astype(jnp.float32), jnp.int32),
         jnp.zeros((pad,), jnp.int32)])
    edata = jnp.stack([src.reshape(E_PAD // SUB, SUB),
                       dst.reshape(E_PAD // SUB, SUB),
                       ewb.reshape(E_PAD // SUB, SUB)], axis=1)
    zeros_init = jnp.zeros((ROWS_PER_TILE, IN_CH), jnp.float32)
    partials = _sc_aggregate(x, edata, zeros_init)
    return _tc_update(partials, x, W_rel, W_root, b_rel, b_root)


# speedup vs baseline: 1.5539x; 1.5539x over previous
import functools

import jax
import jax.numpy as jnp
from jax import lax
from jax.experimental import pallas as pl
from jax.experimental.pallas import tpu as pltpu
from jax.experimental.pallas import tpu_sc as plsc

N_NODES = 100000
IN_CH = 8
OUT_CH = 32

NC = 2
NS = 16
NW = NC * NS
SUB = 128
K = 12
E_W = 100000
CPW = 65
TAIL = E_W - CPW * K * SUB
KT = 2
ROWS_PER_TILE = 6272
N_PAD = NS * ROWS_PER_TILE
XS = 6256
XS_LAST = N_NODES - (NS - 1) * XS


def _sc_mesh():
    return plsc.VectorSubcoreMesh(core_axis_name="c", subcore_axis_name="s")


@functools.partial(
    pl.kernel,
    out_type=jax.ShapeDtypeStruct((NC, N_PAD, IN_CH), jnp.float32),
    mesh=_sc_mesh(),
    scratch_types=[
        pltpu.VMEM_SHARED((N_PAD, IN_CH), jnp.float32),
        pltpu.VMEM_SHARED((N_PAD, IN_CH), jnp.float32),
        pltpu.VMEM((2, K * SUB), jnp.int32),
        pltpu.VMEM((2, K * SUB), jnp.float32),
        pltpu.VMEM((2, K, SUB), jnp.int32),
        pltpu.VMEM((K, SUB, IN_CH), jnp.float32),
        pltpu.VMEM((KT, SUB), jnp.int32),
        pltpu.VMEM((KT, SUB), jnp.int32),
        pltpu.VMEM((KT, SUB), jnp.float32),
        pltpu.SemaphoreType.DMA,
        pltpu.SemaphoreType.DMA,
        pltpu.SemaphoreType.DMA,
        pltpu.SemaphoreType.DMA,
        pltpu.SemaphoreType.DMA,
    ],
    compiler_params=pltpu.CompilerParams(needs_layout_passes=False,
                                         use_tc_tiling_on_sc=False),
)
def _sc_aggregate(x_hbm, src_hbm, dst_hbm, ew_hbm, tsrc_hbm, tdst_hbm,
                  tew_hbm, zero_hbm, out_hbm,
                  x_sh, agg_sh, src_v, ew_v, dst2, rows_v,
                  tsrc_v, tdst_v, tew_v,
                  sem_e, sem_d, sem_g, sem_s, sem_t):
    c = lax.axis_index("c")
    s = lax.axis_index("s")
    wid = c * NS + s
    wbase = wid * E_W

    pltpu.sync_copy(zero_hbm, agg_sh.at[pl.ds(s * ROWS_PER_TILE, ROWS_PER_TILE)])

    @pl.when(s < NS - 1)
    def _():
        pltpu.sync_copy(x_hbm.at[pl.ds(s * XS, XS)], x_sh.at[pl.ds(s * XS, XS)])

    @pl.when(s == NS - 1)
    def _():
        pltpu.sync_copy(x_hbm.at[pl.ds(s * XS, XS_LAST)],
                        x_sh.at[pl.ds(s * XS, XS_LAST)])

    tails = [pltpu.async_copy(tsrc_hbm.at[wid], tsrc_v, sem_t),
             pltpu.async_copy(tdst_hbm.at[wid], tdst_v, sem_t),
             pltpu.async_copy(tew_hbm.at[wid], tew_v, sem_t)]

    plsc.subcore_barrier()

    iota = lax.broadcasted_iota(jnp.int32, (16,), 0)
    cidx = jnp.bitwise_and(iota, 7)
    half = jnp.right_shift(iota, 3)

    def fire_chunk(g, p):
        base = wbase + g * (K * SUB)
        pltpu.async_copy(src_hbm.at[pl.ds(base, K * SUB)], src_v.at[p], sem_e)
        pltpu.async_copy(ew_hbm.at[pl.ds(base, K * SUB)], ew_v.at[p], sem_e)
        for j in range(K):
            pltpu.async_copy(dst_hbm.at[pl.ds(base + j * SUB, SUB)],
                             dst2.at[p, j], sem_d)

    fire_chunk(0, 0)

    def chunk_body(g, carry):
        p = jnp.bitwise_and(g, 1)
        q = 1 - p
        base = wbase + g * (K * SUB)

        pltpu.make_async_copy(src_hbm.at[pl.ds(base, K * SUB)], src_v.at[p],
                              sem_e).wait()
        pltpu.make_async_copy(ew_hbm.at[pl.ds(base, K * SUB)], ew_v.at[p],
                              sem_e).wait()
        for j in range(K):
            pltpu.make_async_copy(dst_hbm.at[pl.ds(base + j * SUB, SUB)],
                                  dst2.at[p, j], sem_d).wait()

        @pl.when(g < CPW - 1)
        def _():
            fire_chunk(g + 1, q)

        gathers = [pltpu.async_copy(
            x_sh.at[src_v.at[p, pl.ds(j * SUB, SUB)]], rows_v.at[j], sem_g)
            for j in range(K)]
        for h in gathers:
            h.wait()

        pv = jnp.full((16,), p, jnp.int32)

        def mul_body(i):
            jv = jnp.full((16,), jnp.right_shift(i, 6), jnp.int32)
            ridx = half + 2 * jnp.bitwise_and(i, 63)
            w = plsc.load_gather(ew_v, [pv, half + 2 * i])
            rows16 = plsc.load_gather(rows_v, [jv, ridx, cidx])
            plsc.store_scatter(rows_v, [jv, ridx, cidx], rows16 * w)

        plsc.parallel_loop(0, K * (SUB // 2), 1, unroll=8)(mul_body)

        scatters = [pltpu.async_copy(rows_v.at[j], agg_sh.at[dst2.at[p, j]],
                                     sem_s, add=True) for j in range(K)]
        for h in scatters:
            h.wait()
        return carry

    lax.fori_loop(0, CPW, chunk_body, 0)

    for h in tails:
        h.wait()
    tgathers = [pltpu.async_copy(x_sh.at[tsrc_v.at[j]], rows_v.at[j], sem_g)
                for j in range(KT)]
    for h in tgathers:
        h.wait()

    def tail_mul(i):
        jv = jnp.full((16,), jnp.right_shift(i, 6), jnp.int32)
        ridx = half + 2 * jnp.bitwise_and(i, 63)
        w = plsc.load_gather(tew_v, [jv, ridx])
        rows16 = plsc.load_gather(rows_v, [jv, ridx, cidx])
        plsc.store_scatter(rows_v, [jv, ridx, cidx], rows16 * w)

    plsc.parallel_loop(0, KT * (SUB // 2), 1, unroll=8)(tail_mul)

    tscat = [pltpu.async_copy(rows_v.at[j], agg_sh.at[tdst_v.at[j]],
                              sem_s, add=True) for j in range(KT)]
    for h in tscat:
        h.wait()

    plsc.subcore_barrier()
    pltpu.sync_copy(agg_sh.at[pl.ds(s * ROWS_PER_TILE, ROWS_PER_TILE)],
                    out_hbm.at[c, pl.ds(s * ROWS_PER_TILE, ROWS_PER_TILE)])


def _tc_body(p_ref, x_ref, m1_ref, m2_ref, b_ref, o_ref):
    a = p_ref[0] + p_ref[1]
    o_ref[...] = (
        jnp.dot(a, m1_ref[...], preferred_element_type=jnp.float32)
        + jnp.dot(x_ref[...], m2_ref[...], preferred_element_type=jnp.float32)
        + b_ref[...]
    )


def _tc_update(partials, x, W_rel, W_root, b_rel, b_root):
    p128 = partials.reshape(NC, N_PAD // 16, 128)
    x128 = x.reshape(N_NODES // 16, 128)
    eye = jnp.eye(16, dtype=jnp.float32)
    m1 = jnp.kron(eye, W_rel)
    m2 = jnp.kron(eye, W_root)
    bias = jnp.tile(b_rel + b_root, 16).reshape(1, 16 * OUT_CH)
    rows = N_NODES // 16
    B = 512
    n_blocks = (rows + B - 1) // B
    out512 = pl.pallas_call(
        _tc_body,
        grid=(n_blocks,),
        in_specs=[
            pl.BlockSpec((NC, B, 128), lambda i: (0, i, 0)),
            pl.BlockSpec((B, 128), lambda i: (i, 0)),
            pl.BlockSpec((128, 16 * OUT_CH), lambda i: (0, 0)),
            pl.BlockSpec((128, 16 * OUT_CH), lambda i: (0, 0)),
            pl.BlockSpec((1, 16 * OUT_CH), lambda i: (0, 0)),
        ],
        out_specs=pl.BlockSpec((B, 16 * OUT_CH), lambda i: (i, 0)),
        out_shape=jax.ShapeDtypeStruct((rows, 16 * OUT_CH), jnp.float32),
    )(p128, x128, m1, m2, bias)
    return out512.reshape(N_NODES, OUT_CH)


def kernel(x, edge_index, edge_weight, W_rel, b_rel, W_root, b_root):
    src = edge_index[0].astype(jnp.int32)
    dst = edge_index[1].astype(jnp.int32)
    ew = edge_weight.astype(jnp.float32)

    tail0 = CPW * K * SUB
    pos = jnp.arange(KT * SUB)
    gidx = jnp.arange(NW)[:, None] * E_W + tail0 + pos[None, :]
    tmask = pos[None, :] < TAIL
    gsafe = jnp.where(tmask, gidx, 0)
    tsrc = jnp.where(tmask, src[gsafe], 0).reshape(NW, KT, SUB)
    tdst = jnp.where(tmask, dst[gsafe], N_NODES).reshape(NW, KT, SUB)
    tew = jnp.where(tmask, ew[gsafe], 0.0).reshape(NW, KT, SUB)

    zeros_init = jnp.zeros((ROWS_PER_TILE, IN_CH), jnp.float32)
    partials = _sc_aggregate(x, src, dst, ew, tsrc, tdst, tew, zeros_init)
    return _tc_update(partials, x, W_rel, W_root, b_rel, b_root)


# speedup vs baseline: 1.7366x; 1.1176x over previous
import functools

import jax
import jax.numpy as jnp
from jax import lax
from jax.experimental import pallas as pl
from jax.experimental.pallas import tpu as pltpu
from jax.experimental.pallas import tpu_sc as plsc

N_NODES = 100000
IN_CH = 8
OUT_CH = 32

NC = 2
NS = 16
NW = NC * NS
SUB = 128
K = 12
E_W = 100000
CPW = 65
TAIL = E_W - CPW * K * SUB
KT = 2
ROWS_PER_TILE = 6272
N_PAD = NS * ROWS_PER_TILE
XS = 6256
XS_LAST = N_NODES - (NS - 1) * XS


def _sc_mesh():
    return plsc.VectorSubcoreMesh(core_axis_name="c", subcore_axis_name="s")


@functools.partial(
    pl.kernel,
    out_type=jax.ShapeDtypeStruct((NC, N_PAD, IN_CH), jnp.float32),
    mesh=_sc_mesh(),
    scratch_types=[
        pltpu.VMEM_SHARED((N_PAD, IN_CH), jnp.float32),
        pltpu.VMEM_SHARED((N_PAD, IN_CH), jnp.float32),
        pltpu.VMEM((2, K * SUB), jnp.int32),
        pltpu.VMEM((2, K * SUB), jnp.float32),
        pltpu.VMEM((2, K, SUB), jnp.int32),
        pltpu.VMEM((K, SUB, IN_CH), jnp.float32),
        pltpu.VMEM((KT, SUB), jnp.int32),
        pltpu.VMEM((KT, SUB), jnp.int32),
        pltpu.VMEM((KT, SUB), jnp.float32),
        pltpu.SemaphoreType.DMA,
        pltpu.SemaphoreType.DMA,
        pltpu.SemaphoreType.DMA,
        pltpu.SemaphoreType.DMA,
        pltpu.SemaphoreType.DMA,
    ],
    compiler_params=pltpu.CompilerParams(needs_layout_passes=False,
                                         use_tc_tiling_on_sc=False),
)
def _sc_aggregate(x_hbm, src_hbm, dst_hbm, ew_hbm, zero_hbm, out_hbm,
                  x_sh, agg_sh, src_v, ew_v, dst2, rows_v,
                  tsrc_v, tdst_v, tew_v,
                  sem_e, sem_d, sem_g, sem_s, sem_t):
    c = lax.axis_index("c")
    s = lax.axis_index("s")
    wid = c * NS + s
    wbase = wid * E_W

    pltpu.sync_copy(zero_hbm, agg_sh.at[pl.ds(s * ROWS_PER_TILE, ROWS_PER_TILE)])

    @pl.when(s < NS - 1)
    def _():
        pltpu.sync_copy(x_hbm.at[pl.ds(s * XS, XS)], x_sh.at[pl.ds(s * XS, XS)])

    @pl.when(s == NS - 1)
    def _():
        pltpu.sync_copy(x_hbm.at[pl.ds(s * XS, XS_LAST)],
                        x_sh.at[pl.ds(s * XS, XS_LAST)])

    tb = wbase + CPW * K * SUB
    tails = [
        pltpu.async_copy(src_hbm.at[pl.ds(tb, SUB)], tsrc_v.at[0], sem_t),
        pltpu.async_copy(src_hbm.at[pl.ds(tb + SUB, TAIL - SUB)],
                         tsrc_v.at[1, pl.ds(0, TAIL - SUB)], sem_t),
        pltpu.async_copy(dst_hbm.at[pl.ds(tb, SUB)], tdst_v.at[0], sem_t),
        pltpu.async_copy(dst_hbm.at[pl.ds(tb + SUB, TAIL - SUB)],
                         tdst_v.at[1, pl.ds(0, TAIL - SUB)], sem_t),
        pltpu.async_copy(ew_hbm.at[pl.ds(tb, SUB)], tew_v.at[0], sem_t),
        pltpu.async_copy(ew_hbm.at[pl.ds(tb + SUB, TAIL - SUB)],
                         tew_v.at[1, pl.ds(0, TAIL - SUB)], sem_t),
    ]
    zi = jnp.zeros((16,), jnp.int32)
    zf = jnp.zeros((16,), jnp.float32)
    dump = jnp.full((16,), N_NODES, jnp.int32)
    for b in range((SUB - (TAIL - SUB)) // 16):
        off = (TAIL - SUB) + 16 * b
        tsrc_v[1, pl.ds(off, 16)] = zi
        tdst_v[1, pl.ds(off, 16)] = dump
        tew_v[1, pl.ds(off, 16)] = zf

    plsc.subcore_barrier()

    iota = lax.broadcasted_iota(jnp.int32, (16,), 0)
    cidx = jnp.bitwise_and(iota, 7)
    half = jnp.right_shift(iota, 3)

    def fire_chunk(g, p):
        base = wbase + g * (K * SUB)
        pltpu.async_copy(src_hbm.at[pl.ds(base, K * SUB)], src_v.at[p], sem_e)
        pltpu.async_copy(ew_hbm.at[pl.ds(base, K * SUB)], ew_v.at[p], sem_e)
        for j in range(K):
            pltpu.async_copy(dst_hbm.at[pl.ds(base + j * SUB, SUB)],
                             dst2.at[p, j], sem_d)

    fire_chunk(0, 0)

    def chunk_body(g, carry):
        p = jnp.bitwise_and(g, 1)
        q = 1 - p
        base = wbase + g * (K * SUB)

        pltpu.make_async_copy(src_hbm.at[pl.ds(base, K * SUB)], src_v.at[p],
                              sem_e).wait()
        pltpu.make_async_copy(ew_hbm.at[pl.ds(base, K * SUB)], ew_v.at[p],
                              sem_e).wait()
        for j in range(K):
            pltpu.make_async_copy(dst_hbm.at[pl.ds(base + j * SUB, SUB)],
                                  dst2.at[p, j], sem_d).wait()

        @pl.when(g < CPW - 1)
        def _():
            fire_chunk(g + 1, q)

        gathers = [pltpu.async_copy(
            x_sh.at[src_v.at[p, pl.ds(j * SUB, SUB)]], rows_v.at[j], sem_g)
            for j in range(K)]
        for h in gathers:
            h.wait()

        pv = jnp.full((16,), p, jnp.int32)

        def mul_body(i):
            jv = jnp.full((16,), jnp.right_shift(i, 6), jnp.int32)
            ridx = half + 2 * jnp.bitwise_and(i, 63)
            w = plsc.load_gather(ew_v, [pv, half + 2 * i])
            rows16 = plsc.load_gather(rows_v, [jv, ridx, cidx])
            plsc.store_scatter(rows_v, [jv, ridx, cidx], rows16 * w)

        plsc.parallel_loop(0, K * (SUB // 2), 1, unroll=8)(mul_body)

        scatters = [pltpu.async_copy(rows_v.at[j], agg_sh.at[dst2.at[p, j]],
                                     sem_s, add=True) for j in range(K)]
        for h in scatters:
            h.wait()
        return carry

    lax.fori_loop(0, CPW, chunk_body, 0)

    for h in tails:
        h.wait()
    tgathers = [pltpu.async_copy(x_sh.at[tsrc_v.at[j]], rows_v.at[j], sem_g)
                for j in range(KT)]
    for h in tgathers:
        h.wait()

    def tail_mul(i):
        jv = jnp.full((16,), jnp.right_shift(i, 6), jnp.int32)
        ridx = half + 2 * jnp.bitwise_and(i, 63)
        w = plsc.load_gather(tew_v, [jv, ridx])
        rows16 = plsc.load_gather(rows_v, [jv, ridx, cidx])
        plsc.store_scatter(rows_v, [jv, ridx, cidx], rows16 * w)

    plsc.parallel_loop(0, KT * (SUB // 2), 1, unroll=8)(tail_mul)

    tscat = [pltpu.async_copy(rows_v.at[j], agg_sh.at[tdst_v.at[j]],
                              sem_s, add=True) for j in range(KT)]
    for h in tscat:
        h.wait()

    plsc.subcore_barrier()
    pltpu.sync_copy(agg_sh.at[pl.ds(s * ROWS_PER_TILE, ROWS_PER_TILE)],
                    out_hbm.at[c, pl.ds(s * ROWS_PER_TILE, ROWS_PER_TILE)])


def _tc_body(p_ref, x_ref, m1_ref, m2_ref, b_ref, o_ref):
    a = p_ref[0] + p_ref[1]
    o_ref[...] = (
        jnp.dot(a, m1_ref[...], preferred_element_type=jnp.float32)
        + jnp.dot(x_ref[...], m2_ref[...], preferred_element_type=jnp.float32)
        + b_ref[...]
    )


def _tc_update(partials, x, W_rel, W_root, b_rel, b_root):
    p128 = partials.reshape(NC, N_PAD // 16, 128)
    x128 = x.reshape(N_NODES // 16, 128)
    eye = jnp.eye(16, dtype=jnp.float32)
    m1 = jnp.kron(eye, W_rel)
    m2 = jnp.kron(eye, W_root)
    bias = jnp.tile(b_rel + b_root, 16).reshape(1, 16 * OUT_CH)
    rows = N_NODES // 16
    B = 512
    n_blocks = (rows + B - 1) // B
    out512 = pl.pallas_call(
        _tc_body,
        grid=(n_blocks,),
        in_specs=[
            pl.BlockSpec((NC, B, 128), lambda i: (0, i, 0)),
            pl.BlockSpec((B, 128), lambda i: (i, 0)),
            pl.BlockSpec((128, 16 * OUT_CH), lambda i: (0, 0)),
            pl.BlockSpec((128, 16 * OUT_CH), lambda i: (0, 0)),
            pl.BlockSpec((1, 16 * OUT_CH), lambda i: (0, 0)),
        ],
        out_specs=pl.BlockSpec((B, 16 * OUT_CH), lambda i: (i, 0)),
        out_shape=jax.ShapeDtypeStruct((rows, 16 * OUT_CH), jnp.float32),
    )(p128, x128, m1, m2, bias)
    return out512.reshape(N_NODES, OUT_CH)


def kernel(x, edge_index, edge_weight, W_rel, b_rel, W_root, b_root):
    src = edge_index[0].astype(jnp.int32)
    dst = edge_index[1].astype(jnp.int32)
    ew = edge_weight.astype(jnp.float32)
    zeros_init = jnp.zeros((ROWS_PER_TILE, IN_CH), jnp.float32)
    partials = _sc_aggregate(x, src, dst, ew, zeros_init)
    return _tc_update(partials, x, W_rel, W_root, b_rel, b_root)


# speedup vs baseline: 2.0073x; 1.1559x over previous
import functools

import jax
import jax.numpy as jnp
from jax import lax
from jax.experimental import pallas as pl
from jax.experimental.pallas import tpu as pltpu
from jax.experimental.pallas import tpu_sc as plsc

N_NODES = 100000
IN_CH = 8
OUT_CH = 32

NC = 2
NS = 16
NW = NC * NS
SUB = 128
K = 12
E_W = 100000
CPW = 65
TAIL = E_W - CPW * K * SUB
KT = 2
ROWS_PER_TILE = 6272
N_PAD = NS * ROWS_PER_TILE
XS = 6256
XS_LAST = N_NODES - (NS - 1) * XS


def _sc_mesh():
    return plsc.VectorSubcoreMesh(core_axis_name="c", subcore_axis_name="s")


@functools.partial(
    pl.kernel,
    out_type=jax.ShapeDtypeStruct((NC, N_PAD, IN_CH), jnp.float32),
    mesh=_sc_mesh(),
    scratch_types=[
        pltpu.VMEM_SHARED((N_PAD, IN_CH), jnp.float32),
        pltpu.VMEM_SHARED((N_PAD, IN_CH), jnp.float32),
        pltpu.VMEM((2, K * SUB), jnp.int32),
        pltpu.VMEM((2, K * SUB), jnp.float32),
        pltpu.VMEM((2, K, SUB), jnp.int32),
        pltpu.VMEM((K, SUB, IN_CH), jnp.float32),
        pltpu.VMEM((KT, SUB), jnp.int32),
        pltpu.VMEM((KT, SUB), jnp.int32),
        pltpu.VMEM((KT, SUB), jnp.float32),
        pltpu.SemaphoreType.DMA,
        pltpu.SemaphoreType.DMA,
        pltpu.SemaphoreType.DMA,
        pltpu.SemaphoreType.DMA,
        pltpu.SemaphoreType.DMA,
    ],
    compiler_params=pltpu.CompilerParams(needs_layout_passes=False,
                                         use_tc_tiling_on_sc=False),
)
def _sc_aggregate(x_hbm, src_hbm, dst_hbm, ew_hbm, zero_hbm, out_hbm,
                  x_sh, agg_sh, src_v, ew_v, dst2, rows_v,
                  tsrc_v, tdst_v, tew_v,
                  sem_e, sem_d, sem_g, sem_s, sem_t):
    c = lax.axis_index("c")
    s = lax.axis_index("s")
    wid = c * NS + s
    wbase = wid * E_W

    pltpu.sync_copy(zero_hbm, agg_sh.at[pl.ds(s * ROWS_PER_TILE, ROWS_PER_TILE)])

    @pl.when(s < NS - 1)
    def _():
        pltpu.sync_copy(x_hbm.at[pl.ds(s * XS, XS)], x_sh.at[pl.ds(s * XS, XS)])

    @pl.when(s == NS - 1)
    def _():
        pltpu.sync_copy(x_hbm.at[pl.ds(s * XS, XS_LAST)],
                        x_sh.at[pl.ds(s * XS, XS_LAST)])

    tb = wbase + CPW * K * SUB
    tails = [
        pltpu.async_copy(src_hbm.at[pl.ds(tb, SUB)], tsrc_v.at[0], sem_t),
        pltpu.async_copy(src_hbm.at[pl.ds(tb + SUB, TAIL - SUB)],
                         tsrc_v.at[1, pl.ds(0, TAIL - SUB)], sem_t),
        pltpu.async_copy(dst_hbm.at[pl.ds(tb, SUB)], tdst_v.at[0], sem_t),
        pltpu.async_copy(dst_hbm.at[pl.ds(tb + SUB, TAIL - SUB)],
                         tdst_v.at[1, pl.ds(0, TAIL - SUB)], sem_t),
        pltpu.async_copy(ew_hbm.at[pl.ds(tb, SUB)], tew_v.at[0], sem_t),
        pltpu.async_copy(ew_hbm.at[pl.ds(tb + SUB, TAIL - SUB)],
                         tew_v.at[1, pl.ds(0, TAIL - SUB)], sem_t),
    ]
    zi = jnp.zeros((16,), jnp.int32)
    zf = jnp.zeros((16,), jnp.float32)
    dump = jnp.full((16,), N_NODES, jnp.int32)
    for b in range((SUB - (TAIL - SUB)) // 16):
        off = (TAIL - SUB) + 16 * b
        tsrc_v[1, pl.ds(off, 16)] = zi
        tdst_v[1, pl.ds(off, 16)] = dump
        tew_v[1, pl.ds(off, 16)] = zf

    plsc.subcore_barrier()

    iota = lax.broadcasted_iota(jnp.int32, (16,), 0)
    cidx = jnp.bitwise_and(iota, 7)
    half = jnp.right_shift(iota, 3)

    def fire_chunk(g, p):
        base = wbase + g * (K * SUB)
        pltpu.async_copy(src_hbm.at[pl.ds(base, K * SUB)], src_v.at[p], sem_e)
        pltpu.async_copy(ew_hbm.at[pl.ds(base, K * SUB)], ew_v.at[p], sem_e)
        for j in range(K):
            pltpu.async_copy(dst_hbm.at[pl.ds(base + j * SUB, SUB)],
                             dst2.at[p, j], sem_d)

    fire_chunk(0, 0)

    def chunk_body(g, carry):
        p = jnp.bitwise_and(g, 1)
        q = 1 - p
        base = wbase + g * (K * SUB)

        pltpu.make_async_copy(src_hbm.at[pl.ds(base, K * SUB)], src_v.at[p],
                              sem_e).wait()
        pltpu.make_async_copy(ew_hbm.at[pl.ds(base, K * SUB)], ew_v.at[p],
                              sem_e).wait()
        for j in range(K):
            pltpu.make_async_copy(dst_hbm.at[pl.ds(base + j * SUB, SUB)],
                                  dst2.at[p, j], sem_d).wait()

        @pl.when(g < CPW - 1)
        def _():
            fire_chunk(g + 1, q)

        gathers = [pltpu.async_copy(
            x_sh.at[src_v.at[p, pl.ds(j * SUB, SUB)]], rows_v.at[j], sem_g)
            for j in range(K)]
        pv = jnp.full((16,), p, jnp.int32)
        scatters = []
        for j in range(K):
            gathers[j].wait()
            jv = jnp.full((16,), j, jnp.int32)

            def mul_body(i, jv=jv, j=j):
                ridx = half + 2 * i
                w = plsc.load_gather(ew_v, [pv, j * SUB + ridx])
                rows16 = plsc.load_gather(rows_v, [jv, ridx, cidx])
                plsc.store_scatter(rows_v, [jv, ridx, cidx], rows16 * w)

            plsc.parallel_loop(0, SUB // 2, 1, unroll=8)(mul_body)
            scatters.append(pltpu.async_copy(
                rows_v.at[j], agg_sh.at[dst2.at[p, j]], sem_s, add=True))
        for h in scatters:
            h.wait()
        return carry

    lax.fori_loop(0, CPW, chunk_body, 0)

    for h in tails:
        h.wait()
    tgathers = [pltpu.async_copy(x_sh.at[tsrc_v.at[j]], rows_v.at[j], sem_g)
                for j in range(KT)]
    for h in tgathers:
        h.wait()

    def tail_mul(i):
        jv = jnp.full((16,), jnp.right_shift(i, 6), jnp.int32)
        ridx = half + 2 * jnp.bitwise_and(i, 63)
        w = plsc.load_gather(tew_v, [jv, ridx])
        rows16 = plsc.load_gather(rows_v, [jv, ridx, cidx])
        plsc.store_scatter(rows_v, [jv, ridx, cidx], rows16 * w)

    plsc.parallel_loop(0, KT * (SUB // 2), 1, unroll=8)(tail_mul)

    tscat = [pltpu.async_copy(rows_v.at[j], agg_sh.at[tdst_v.at[j]],
                              sem_s, add=True) for j in range(KT)]
    for h in tscat:
        h.wait()

    plsc.subcore_barrier()
    pltpu.sync_copy(agg_sh.at[pl.ds(s * ROWS_PER_TILE, ROWS_PER_TILE)],
                    out_hbm.at[c, pl.ds(s * ROWS_PER_TILE, ROWS_PER_TILE)])


def _tc_body(p_ref, x_ref, m1_ref, m2_ref, b_ref, o_ref):
    a = p_ref[0] + p_ref[1]
    o_ref[...] = (
        jnp.dot(a, m1_ref[...], preferred_element_type=jnp.float32)
        + jnp.dot(x_ref[...], m2_ref[...], preferred_element_type=jnp.float32)
        + b_ref[...]
    )


def _tc_update(partials, x, W_rel, W_root, b_rel, b_root):
    p128 = partials.reshape(NC, N_PAD // 16, 128)
    x128 = x.reshape(N_NODES // 16, 128)
    eye = jnp.eye(16, dtype=jnp.float32)
    m1 = jnp.kron(eye, W_rel)
    m2 = jnp.kron(eye, W_root)
    bias = jnp.tile(b_rel + b_root, 16).reshape(1, 16 * OUT_CH)
    rows = N_NODES // 16
    B = 512
    n_blocks = (rows + B - 1) // B
    out512 = pl.pallas_call(
        _tc_body,
        grid=(n_blocks,),
        in_specs=[
            pl.BlockSpec((NC, B, 128), lambda i: (0, i, 0)),
            pl.BlockSpec((B, 128), lambda i: (i, 0)),
            pl.BlockSpec((128, 16 * OUT_CH), lambda i: (0, 0)),
            pl.BlockSpec((128, 16 * OUT_CH), lambda i: (0, 0)),
            pl.BlockSpec((1, 16 * OUT_CH), lambda i: (0, 0)),
        ],
        out_specs=pl.BlockSpec((B, 16 * OUT_CH), lambda i: (i, 0)),
        out_shape=jax.ShapeDtypeStruct((rows, 16 * OUT_CH), jnp.float32),
    )(p128, x128, m1, m2, bias)
    return out512.reshape(N_NODES, OUT_CH)


def kernel(x, edge_index, edge_weight, W_rel, b_rel, W_root, b_root):
    src = edge_index[0].astype(jnp.int32)
    dst = edge_index[1].astype(jnp.int32)
    ew = edge_weight.astype(jnp.float32)
    zeros_init = jnp.zeros((ROWS_PER_TILE, IN_CH), jnp.float32)
    partials = _sc_aggregate(x, src, dst, ew, zeros_init)
    return _tc_update(partials, x, W_rel, W_root, b_rel, b_root)
